# baseline (speedup 1.0000x reference)
import functools

import jax
import jax.numpy as jnp
from jax import lax
from jax.experimental import pallas as pl
from jax.experimental.pallas import tpu as pltpu
from jax.experimental.pallas import tpu_sc as plsc

N = 50000
DIM = 128
K = 1024

BLOCK = 5000
CHUNKS = (15000, 15000, 10000, 10000)
SC_WORKERS = 32


def _vq_compute(z, cbn, sq, i, nblocks, dist_ref, idx_ref, loss_in_ref,
                loss_ref, last):
    zn = z / (jnp.sqrt(jnp.sum(z * z, axis=1, keepdims=True)) + 1e-12)
    d = lax.dot_general(zn, cbn, (((1,), (1,)), ((), ())),
                        preferred_element_type=jnp.float32)
    dist_ref[...] = d
    maxv = jnp.max(d, axis=1)
    kio = lax.broadcasted_iota(jnp.int32, (BLOCK, K), 1)
    idx = jnp.min(jnp.where(d == maxv[:, None], kio, K), axis=1)
    idx_ref[0, 0, :] = idx
    onehot = (kio == idx[:, None]).astype(jnp.float32)
    qsq = lax.dot_general(onehot, sq, (((1,), (0,)), ((), ())),
                          preferred_element_type=jnp.float32)[:, 0]
    znsq = jnp.sum(zn * zn, axis=1)
    s = jnp.sum(qsq + znsq - 2.0 * maxv)

    @pl.when(i == 0)
    def _init():
        loss_ref[0, 0] = loss_in_ref[0, 0]

    loss_ref[0, 0] += s

    if last:
        @pl.when(i == nblocks - 1)
        def _fin():
            loss_ref[0, 0] = loss_ref[0, 0] * (0.25 / (N * DIM))


def _make_tc_first(nblocks, last):
    def body(z_ref, cb_ref, loss_in_ref, dist_ref, idx_ref, loss_ref,
             cbn_ref, sq_ref, cbn_s, sq_s):
        i = pl.program_id(0)

        @pl.when(i == 0)
        def _norm():
            cb = cb_ref[...]
            nrm = jnp.sqrt(jnp.sum(cb * cb, axis=1, keepdims=True)) + 1e-12
            cbn = cb / nrm
            cbn_s[...] = cbn
            cbn_ref[...] = cbn
            sqv = jnp.sum(cbn * cbn, axis=1, keepdims=True)
            sq_s[...] = sqv
            sq_ref[...] = sqv

        _vq_compute(z_ref[...], cbn_s[...], sq_s[...], i, nblocks,
                    dist_ref, idx_ref, loss_in_ref, loss_ref, last)

    return functools.partial(
        pl.pallas_call,
        body,
        grid=(nblocks,),
        in_specs=[
            pl.BlockSpec((BLOCK, DIM), lambda i: (i, 0)),
            pl.BlockSpec((K, DIM), lambda i: (0, 0)),
            pl.BlockSpec((1, 1), lambda i: (0, 0), memory_space=pltpu.SMEM),
        ],
        out_specs=[
            pl.BlockSpec((BLOCK, K), lambda i: (i, 0)),
            pl.BlockSpec((1, 1, BLOCK), lambda i: (i, 0, 0)),
            pl.BlockSpec((1, 1), lambda i: (0, 0), memory_space=pltpu.SMEM),
            pl.BlockSpec((K, DIM), lambda i: (0, 0)),
            pl.BlockSpec((K, 1), lambda i: (0, 0)),
        ],
        out_shape=[
            jax.ShapeDtypeStruct((N, K), jnp.float32),
            jax.ShapeDtypeStruct((nblocks, 1, BLOCK), jnp.int32),
            jax.ShapeDtypeStruct((1, 1), jnp.float32),
            jax.ShapeDtypeStruct((K, DIM), jnp.float32),
            jax.ShapeDtypeStruct((K, 1), jnp.float32),
        ],
        scratch_shapes=[
            pltpu.VMEM((K, DIM), jnp.float32),
            pltpu.VMEM((K, 1), jnp.float32),
        ],
    )()


def _make_tc_chunk(noff, nblocks, last):
    def body(z_ref, cbn_ref, sq_ref, loss_in_ref, _dist_alias,
             dist_ref, idx_ref, loss_ref):
        i = pl.program_id(0)
        _vq_compute(z_ref[...], cbn_ref[...], sq_ref[...], i, nblocks,
                    dist_ref, idx_ref, loss_in_ref, loss_ref, last)

    return functools.partial(
        pl.pallas_call,
        body,
        grid=(nblocks,),
        in_specs=[
            pl.BlockSpec((BLOCK, DIM), lambda i, o=noff: (o + i, 0)),
            pl.BlockSpec((K, DIM), lambda i: (0, 0)),
            pl.BlockSpec((K, 1), lambda i: (0, 0)),
            pl.BlockSpec((1, 1), lambda i: (0, 0), memory_space=pltpu.SMEM),
            pl.BlockSpec(memory_space=pl.ANY),
        ],
        out_specs=[
            pl.BlockSpec((BLOCK, K), lambda i, o=noff: (o + i, 0)),
            pl.BlockSpec((1, 1, BLOCK), lambda i: (i, 0, 0)),
            pl.BlockSpec((1, 1), lambda i: (0, 0), memory_space=pltpu.SMEM),
        ],
        out_shape=[
            jax.ShapeDtypeStruct((N, K), jnp.float32),
            jax.ShapeDtypeStruct((nblocks, 1, BLOCK), jnp.int32),
            jax.ShapeDtypeStruct((1, 1), jnp.float32),
        ],
        input_output_aliases={4: 0},
    )()


def _sc_split(nrows):
    base = (nrows // SC_WORKERS) // 8 * 8
    nplus = (nrows - base * SC_WORKERS) // 8
    return base, nplus


def _sc_gather_body(chunk_start, base, nplus, cbn_hbm, idx_hbm, q_hbm,
                    idx_v, rows_v, sem):
    wid = lax.axis_index("s") * 2 + lax.axis_index("c")
    off = base * wid + 8 * jnp.minimum(wid, nplus)

    @pl.when(wid < nplus)
    def _hi():
        pltpu.sync_copy(idx_hbm.at[pl.ds(off, base + 8)], idx_v)
        pltpu.async_copy(cbn_hbm.at[idx_v], rows_v, sem).wait()
        pltpu.sync_copy(rows_v, q_hbm.at[pl.ds(chunk_start + off, base + 8)])

    @pl.when(wid >= nplus)
    def _lo():
        iv = idx_v.at[pl.ds(0, base)]
        rv = rows_v.at[pl.ds(0, base)]
        pltpu.sync_copy(idx_hbm.at[pl.ds(off, base)], iv)
        pltpu.async_copy(cbn_hbm.at[iv], rv, sem).wait()
        pltpu.sync_copy(rv, q_hbm.at[pl.ds(chunk_start + off, base)])


@functools.lru_cache(maxsize=None)
def _make_sc_gather(chunk_start, nrows, alloc):
    base, nplus = _sc_split(nrows)
    mesh = plsc.VectorSubcoreMesh(core_axis_name="c", subcore_axis_name="s")
    out_type = jax.ShapeDtypeStruct((N, DIM), jnp.float32) if alloc else ()
    body = functools.partial(_sc_gather_body, chunk_start, base, nplus)
    return pl.kernel(
        body,
        mesh=mesh,
        out_type=out_type,
        scratch_types=[
            pltpu.VMEM((base + 8,), jnp.int32),
            pltpu.VMEM((base + 8, DIM), jnp.float32),
            pltpu.SemaphoreType.DMA,
        ],
    )


def kernel(z, codebook):
    loss = jnp.zeros((1, 1), jnp.float32)
    dist = cbn = sq = None
    idx_parts = []
    sc_calls = []
    noff = 0
    for j, nrows in enumerate(CHUNKS):
        nblocks = nrows // BLOCK
        last = j == len(CHUNKS) - 1
        if j == 0:
            dist, idx3, loss, cbn, sq = _make_tc_first(nblocks, last)(
                z, codebook, loss)
        else:
            dist, idx3, loss = _make_tc_chunk(noff, nblocks, last)(
                z, cbn, sq, loss, dist)
        idx_c = idx3.reshape(nrows)
        idx_parts.append(idx_c)
        sc_calls.append((noff * BLOCK, nrows, idx_c))
        noff += nblocks

    start0, nrows0, idx0 = sc_calls[0]
    q0 = _make_sc_gather(start0, nrows0, True)(cbn, idx0)
    qref = jax.new_ref(q0)
    for start, nrows, idx_c in sc_calls[1:]:
        _make_sc_gather(start, nrows, False)(cbn, idx_c, qref)
    quant_st = qref[...]

    idx = jnp.concatenate(idx_parts)
    return quant_st, idx, loss[0, 0], dist, cbn


# speedup vs baseline: 1.0422x; 1.0422x over previous
import functools

import jax
import jax.numpy as jnp
from jax import lax
from jax.experimental import pallas as pl
from jax.experimental.pallas import tpu as pltpu
from jax.experimental.pallas import tpu_sc as plsc

N = 50000
DIM = 128
K = 1024

BLOCK = 5000
CHUNKS = (15000, 15000, 15000, 5000)
SC_WORKERS = 32


def _norm_body(cb_ref, cbn_ref, sq_ref):
    cb = cb_ref[...]
    nrm = jnp.sqrt(jnp.sum(cb * cb, axis=1, keepdims=True)) + 1e-12
    cbn = cb / nrm
    cbn_ref[...] = cbn
    sq_ref[...] = jnp.sum(cbn * cbn, axis=1, keepdims=True)


def _vq_compute(z, cbn, sq, i, nblocks, dist_ref, idx_ref, loss_in_ref,
                loss_ref, last):
    zn = z / (jnp.sqrt(jnp.sum(z * z, axis=1, keepdims=True)) + 1e-12)
    d = lax.dot_general(zn, cbn, (((1,), (1,)), ((), ())),
                        preferred_element_type=jnp.float32)
    dist_ref[...] = d
    maxv = jnp.max(d, axis=1)
    kio = lax.broadcasted_iota(jnp.int32, (BLOCK, K), 1)
    idx = jnp.min(jnp.where(d == maxv[:, None], kio, K), axis=1)
    idx_ref[0, 0, :] = idx
    onehot = (kio == idx[:, None]).astype(jnp.float32)
    qsq = lax.dot_general(onehot, sq, (((1,), (0,)), ((), ())),
                          preferred_element_type=jnp.float32)[:, 0]
    znsq = jnp.sum(zn * zn, axis=1)
    s = jnp.sum(qsq + znsq - 2.0 * maxv)

    @pl.when(i == 0)
    def _init():
        loss_ref[0, 0] = loss_in_ref[0, 0]

    loss_ref[0, 0] += s

    if last:
        @pl.when(i == nblocks - 1)
        def _fin():
            loss_ref[0, 0] = loss_ref[0, 0] * (0.25 / (N * DIM))


def _make_tc_chunk(noff, nblocks, first, last):
    def body(*refs):
        if first:
            z_ref, cbn_ref, sq_ref, loss_in_ref, dist_ref, idx_ref, loss_ref = refs
        else:
            (z_ref, cbn_ref, sq_ref, loss_in_ref, _dist_alias,
             dist_ref, idx_ref, loss_ref) = refs
        i = pl.program_id(0)
        _vq_compute(z_ref[...], cbn_ref[...], sq_ref[...], i, nblocks,
                    dist_ref, idx_ref, loss_in_ref, loss_ref, last)

    in_specs = [
        pl.BlockSpec((BLOCK, DIM), lambda i, o=noff: (o + i, 0)),
        pl.BlockSpec((K, DIM), lambda i: (0, 0)),
        pl.BlockSpec((K, 1), lambda i: (0, 0)),
        pl.BlockSpec((1, 1), lambda i: (0, 0), memory_space=pltpu.SMEM),
    ]
    aliases = {}
    if not first:
        in_specs.append(pl.BlockSpec(memory_space=pl.ANY))
        aliases = {4: 0}
    return functools.partial(
        pl.pallas_call,
        body,
        grid=(nblocks,),
        in_specs=in_specs,
        out_specs=[
            pl.BlockSpec((BLOCK, K), lambda i, o=noff: (o + i, 0)),
            pl.BlockSpec((1, 1, BLOCK), lambda i: (i, 0, 0)),
            pl.BlockSpec((1, 1), lambda i: (0, 0), memory_space=pltpu.SMEM),
        ],
        out_shape=[
            jax.ShapeDtypeStruct((N, K), jnp.float32),
            jax.ShapeDtypeStruct((nblocks, 1, BLOCK), jnp.int32),
            jax.ShapeDtypeStruct((1, 1), jnp.float32),
        ],
        input_output_aliases=aliases,
    )()


def _sc_split(nrows):
    base = (nrows // SC_WORKERS) // 8 * 8
    nplus = (nrows - base * SC_WORKERS) // 8
    return base, nplus


def _sc_gather_body(chunk_start, base, nplus, cbn_hbm, idx_hbm, q_hbm,
                    idx_v, rows_v, sem):
    wid = lax.axis_index("s") * 2 + lax.axis_index("c")
    off = base * wid + 8 * jnp.minimum(wid, nplus)

    @pl.when(wid < nplus)
    def _hi():
        pltpu.sync_copy(idx_hbm.at[pl.ds(off, base + 8)], idx_v)
        pltpu.async_copy(cbn_hbm.at[idx_v], rows_v, sem).wait()
        pltpu.sync_copy(rows_v, q_hbm.at[pl.ds(chunk_start + off, base + 8)])

    @pl.when(wid >= nplus)
    def _lo():
        iv = idx_v.at[pl.ds(0, base)]
        rv = rows_v.at[pl.ds(0, base)]
        pltpu.sync_copy(idx_hbm.at[pl.ds(off, base)], iv)
        pltpu.async_copy(cbn_hbm.at[iv], rv, sem).wait()
        pltpu.sync_copy(rv, q_hbm.at[pl.ds(chunk_start + off, base)])


@functools.lru_cache(maxsize=None)
def _make_sc_gather(chunk_start, nrows, alloc):
    base, nplus = _sc_split(nrows)
    mesh = plsc.VectorSubcoreMesh(core_axis_name="c", subcore_axis_name="s")
    out_type = jax.ShapeDtypeStruct((N, DIM), jnp.float32) if alloc else ()
    body = functools.partial(_sc_gather_body, chunk_start, base, nplus)
    return pl.kernel(
        body,
        mesh=mesh,
        out_type=out_type,
        scratch_types=[
            pltpu.VMEM((base + 8,), jnp.int32),
            pltpu.VMEM((base + 8, DIM), jnp.float32),
            pltpu.SemaphoreType.DMA,
        ],
    )


def kernel(z, codebook):
    cbn, sq = pl.pallas_call(
        _norm_body,
        out_shape=[
            jax.ShapeDtypeStruct((K, DIM), jnp.float32),
            jax.ShapeDtypeStruct((K, 1), jnp.float32),
        ],
    )(codebook)

    loss = jnp.zeros((1, 1), jnp.float32)
    dist = None
    idx_parts = []
    sc_calls = []
    noff = 0
    for j, nrows in enumerate(CHUNKS):
        nblocks = nrows // BLOCK
        first, last = j == 0, j == len(CHUNKS) - 1
        tc = _make_tc_chunk(noff, nblocks, first, last)
        if first:
            dist, idx3, loss = tc(z, cbn, sq, loss)
        else:
            dist, idx3, loss = tc(z, cbn, sq, loss, dist)
        idx_c = idx3.reshape(nrows)
        idx_parts.append(idx_c)
        sc_calls.append((noff * BLOCK, nrows, idx_c))
        noff += nblocks

    start0, nrows0, idx0 = sc_calls[0]
    q0 = _make_sc_gather(start0, nrows0, True)(cbn, idx0)
    qref = jax.new_ref(q0)
    for start, nrows, idx_c in sc_calls[1:]:
        _make_sc_gather(start, nrows, False)(cbn, idx_c, qref)
    quant_st = qref[...]

    idx = jnp.concatenate(idx_parts)
    return quant_st, idx, loss[0, 0], dist, cbn


# speedup vs baseline: 1.0575x; 1.0147x over previous
import functools

import jax
import jax.numpy as jnp
from jax import lax
from jax.experimental import pallas as pl
from jax.experimental.pallas import tpu as pltpu
from jax.experimental.pallas import tpu_sc as plsc

N = 50000
DIM = 128
K = 1024

BLOCK = 5000
CHUNKS = (20000, 15000, 10000, 5000)
SC_WORKERS = 32


def _norm_body(cb_ref, cbn_ref, sq_ref):
    cb = cb_ref[...]
    nrm = jnp.sqrt(jnp.sum(cb * cb, axis=1, keepdims=True)) + 1e-12
    cbn = cb / nrm
    cbn_ref[...] = cbn
    sq_ref[...] = jnp.sum(cbn * cbn, axis=1, keepdims=True)


def _vq_compute(z, cbn, sq, i, nblocks, dist_ref, idx_ref, loss_in_ref,
                loss_ref, last):
    zn = z / (jnp.sqrt(jnp.sum(z * z, axis=1, keepdims=True)) + 1e-12)
    d = lax.dot_general(zn, cbn, (((1,), (1,)), ((), ())),
                        preferred_element_type=jnp.float32)
    dist_ref[...] = d
    maxv = jnp.max(d, axis=1)
    kio = lax.broadcasted_iota(jnp.int32, (BLOCK, K), 1)
    idx = jnp.min(jnp.where(d == maxv[:, None], kio, K), axis=1)
    idx_ref[0, 0, :] = idx
    onehot = (kio == idx[:, None]).astype(jnp.float32)
    qsq = lax.dot_general(onehot, sq, (((1,), (0,)), ((), ())),
                          preferred_element_type=jnp.float32)[:, 0]
    znsq = jnp.sum(zn * zn, axis=1)
    s = jnp.sum(qsq + znsq - 2.0 * maxv)

    @pl.when(i == 0)
    def _init():
        loss_ref[0, 0] = loss_in_ref[0, 0]

    loss_ref[0, 0] += s

    if last:
        @pl.when(i == nblocks - 1)
        def _fin():
            loss_ref[0, 0] = loss_ref[0, 0] * (0.25 / (N * DIM))


def _make_tc_chunk(noff, nblocks, first, last):
    def body(*refs):
        if first:
            z_ref, cbn_ref, sq_ref, loss_in_ref, dist_ref, idx_ref, loss_ref = refs
        else:
            (z_ref, cbn_ref, sq_ref, loss_in_ref, _dist_alias,
             dist_ref, idx_ref, loss_ref) = refs
        i = pl.program_id(0)
        _vq_compute(z_ref[...], cbn_ref[...], sq_ref[...], i, nblocks,
                    dist_ref, idx_ref, loss_in_ref, loss_ref, last)

    in_specs = [
        pl.BlockSpec((BLOCK, DIM), lambda i, o=noff: (o + i, 0)),
        pl.BlockSpec((K, DIM), lambda i: (0, 0)),
        pl.BlockSpec((K, 1), lambda i: (0, 0)),
        pl.BlockSpec((1, 1), lambda i: (0, 0), memory_space=pltpu.SMEM),
    ]
    aliases = {}
    if not first:
        in_specs.append(pl.BlockSpec(memory_space=pl.ANY))
        aliases = {4: 0}
    return functools.partial(
        pl.pallas_call,
        body,
        grid=(nblocks,),
        in_specs=in_specs,
        out_specs=[
            pl.BlockSpec((BLOCK, K), lambda i, o=noff: (o + i, 0)),
            pl.BlockSpec((1, 1, BLOCK), lambda i: (i, 0, 0)),
            pl.BlockSpec((1, 1), lambda i: (0, 0), memory_space=pltpu.SMEM),
        ],
        out_shape=[
            jax.ShapeDtypeStruct((N, K), jnp.float32),
            jax.ShapeDtypeStruct((nblocks, 1, BLOCK), jnp.int32),
            jax.ShapeDtypeStruct((1, 1), jnp.float32),
        ],
        input_output_aliases=aliases,
    )()


def _sc_split(nrows):
    base = (nrows // SC_WORKERS) // 8 * 8
    nplus = (nrows - base * SC_WORKERS) // 8
    return base, nplus


def _sc_gather_body(chunk_start, base, nplus, cbn_hbm, idx_hbm, q_hbm,
                    idx_v, rows_v, sem):
    wid = lax.axis_index("s") * 2 + lax.axis_index("c")
    off = base * wid + 8 * jnp.minimum(wid, nplus)

    @pl.when(wid < nplus)
    def _hi():
        pltpu.sync_copy(idx_hbm.at[pl.ds(off, base + 8)], idx_v)
        pltpu.async_copy(cbn_hbm.at[idx_v], rows_v, sem).wait()
        pltpu.sync_copy(rows_v, q_hbm.at[pl.ds(chunk_start + off, base + 8)])

    @pl.when(wid >= nplus)
    def _lo():
        iv = idx_v.at[pl.ds(0, base)]
        rv = rows_v.at[pl.ds(0, base)]
        pltpu.sync_copy(idx_hbm.at[pl.ds(off, base)], iv)
        pltpu.async_copy(cbn_hbm.at[iv], rv, sem).wait()
        pltpu.sync_copy(rv, q_hbm.at[pl.ds(chunk_start + off, base)])


@functools.lru_cache(maxsize=None)
def _make_sc_gather(chunk_start, nrows, alloc):
    base, nplus = _sc_split(nrows)
    mesh = plsc.VectorSubcoreMesh(core_axis_name="c", subcore_axis_name="s")
    out_type = jax.ShapeDtypeStruct((N, DIM), jnp.float32) if alloc else ()
    body = functools.partial(_sc_gather_body, chunk_start, base, nplus)
    return pl.kernel(
        body,
        mesh=mesh,
        out_type=out_type,
        scratch_types=[
            pltpu.VMEM((base + 8,), jnp.int32),
            pltpu.VMEM((base + 8, DIM), jnp.float32),
            pltpu.SemaphoreType.DMA,
        ],
    )


def kernel(z, codebook):
    cbn, sq = pl.pallas_call(
        _norm_body,
        out_shape=[
            jax.ShapeDtypeStruct((K, DIM), jnp.float32),
            jax.ShapeDtypeStruct((K, 1), jnp.float32),
        ],
    )(codebook)

    loss = jnp.zeros((1, 1), jnp.float32)
    dist = None
    idx_parts = []
    sc_calls = []
    noff = 0
    for j, nrows in enumerate(CHUNKS):
        nblocks = nrows // BLOCK
        first, last = j == 0, j == len(CHUNKS) - 1
        tc = _make_tc_chunk(noff, nblocks, first, last)
        if first:
            dist, idx3, loss = tc(z, cbn, sq, loss)
        else:
            dist, idx3, loss = tc(z, cbn, sq, loss, dist)
        idx_c = idx3.reshape(nrows)
        idx_parts.append(idx_c)
        sc_calls.append((noff * BLOCK, nrows, idx_c))
        noff += nblocks

    start0, nrows0, idx0 = sc_calls[0]
    q0 = _make_sc_gather(start0, nrows0, True)(cbn, idx0)
    qref = jax.new_ref(q0)
    for start, nrows, idx_c in sc_calls[1:]:
        _make_sc_gather(start, nrows, False)(cbn, idx_c, qref)
    quant_st = qref[...]

    idx = jnp.concatenate(idx_parts)
    return quant_st, idx, loss[0, 0], dist, cbn
